# Initial kernel scaffold; baseline (speedup 1.0000x reference)
import jax
import jax.numpy as jnp
from jax.experimental import pallas as pl


def kernel(h, e, p, edge_index, batch, Wh, bh, We, be, Wp, bp, lin_W, lin_b, hps_W, hps_b, hpr_W, hpr_b, p1_W, p1_b, p2_W, p2_b, bn_w, bn_b, r1_W, r1_b, r2_W, r2_b, r3_W, r3_b):
    raise NotImplementedError("write your pallas kernel here")


# speedup vs baseline: 1.5173x; 1.5173x over previous
import functools

import jax
import jax.numpy as jnp
from jax import lax
from jax.experimental import pallas as pl

N = 10000
E = 160000
FEAT = 128
POS = 16
EF = 16
H = 64
L = 2
G = 128

EBLK = 6400


def _dgT(w, x):
    return lax.dot_general(w, x, (((0,), (0,)), ((), ())),
                           preferred_element_type=jnp.float32)


def _t0_body(h_ref, wh_ref, bh_ref, p_ref, wp_ref, bp_ref, ht_ref, pt_ref):
    h = h_ref[...]
    ht_ref[...] = lax.dot_general(wh_ref[...], h, (((0,), (1,)), ((), ())),
                                  preferred_element_type=jnp.float32) + bh_ref[...]
    p = p_ref[...]
    pt_ref[...] = lax.dot_general(wp_ref[...], p, (((0,), (1,)), ((), ())),
                                  preferred_element_type=jnp.float32) + bp_ref[...]


def _t0(h, Wh, bh, p, Wp, bp):
    return pl.pallas_call(
        _t0_body,
        out_shape=[
            jax.ShapeDtypeStruct((H, N), jnp.float32),
            jax.ShapeDtypeStruct((H, N), jnp.float32),
        ],
    )(h, Wh, bh.reshape(H, 1), p, Wp, bp.reshape(H, 1))


def _t1_body(ht_ref, pt_ref, linw_ref, linb_ref, hpsa_ref, hpsb_ref, hpsbias_ref,
             hpra_ref, hprb_ref, hprbias_ref, p1w_ref, p1b_ref, p2w_ref, p2b_ref,
             u_ref, vs_ref, vr_ref, q1_ref, q2_ref):
    ht = ht_ref[...]
    pt = pt_ref[...]
    u_ref[...] = _dgT(linw_ref[...], ht) + linb_ref[...]
    vs_ref[...] = _dgT(hpsa_ref[...], ht) + _dgT(hpsb_ref[...], pt) + hpsbias_ref[...]
    vr_ref[...] = _dgT(hpra_ref[...], ht) + _dgT(hprb_ref[...], pt) + hprbias_ref[...]
    q1_ref[...] = _dgT(p1w_ref[...], pt) + p1b_ref[...]
    q2_ref[...] = _dgT(p2w_ref[...], pt) + p2b_ref[...]


def _t1(ht, pt, lin_W, lin_b, hps_W, hps_b, hpr_W, hpr_b, p1_W, p1_b, p2_W, p2_b):
    return pl.pallas_call(
        _t1_body,
        out_shape=[jax.ShapeDtypeStruct((H, N), jnp.float32)] * 5,
    )(ht, pt, lin_W, lin_b.reshape(H, 1),
      hps_W[:H], hps_W[H:], hps_b.reshape(H, 1),
      hpr_W[:H], hpr_W[H:], hpr_b.reshape(H, 1),
      p1_W, p1_b.reshape(H, 1), p2_W, p2_b.reshape(H, 1))


def _te0_body(e_ref, a_ref, c_ref, out_ref):
    out_ref[...] = lax.dot_general(a_ref[...], e_ref[...], (((0,), (1,)), ((), ())),
                                   preferred_element_type=jnp.float32) + c_ref[...]


def _te0(e_raw, A0, c0):
    grid = (E // EBLK,)
    return pl.pallas_call(
        _te0_body,
        grid=grid,
        in_specs=[
            pl.BlockSpec((EBLK, EF), lambda i: (i, 0)),
            pl.BlockSpec((EF, H), lambda i: (0, 0)),
            pl.BlockSpec((H, 1), lambda i: (0, 0)),
        ],
        out_specs=pl.BlockSpec((H, EBLK), lambda i: (0, i)),
        out_shape=jax.ShapeDtypeStruct((H, E), jnp.float32),
    )(e_raw, A0, c0.reshape(H, 1))


def _te1_body(e_ref, eta_ref, a_ref, lw_ref, c_ref, bnm_ref, out_ref):
    eta = eta_ref[...]
    mu = bnm_ref[0:1, :].reshape(H, 1)
    sc = bnm_ref[1:2, :].reshape(H, 1)
    bb = bnm_ref[2:3, :].reshape(H, 1)
    r = jnp.maximum((eta - mu) * sc + bb, 0.0)
    out_ref[...] = (
        lax.dot_general(a_ref[...], e_ref[...], (((0,), (1,)), ((), ())),
                        preferred_element_type=jnp.float32)
        + _dgT(lw_ref[...], r) + c_ref[...])


def _te1(e_raw, eta0, A1, lin_W1, c1, bnm):
    grid = (E // EBLK,)
    return pl.pallas_call(
        _te1_body,
        grid=grid,
        in_specs=[
            pl.BlockSpec((EBLK, EF), lambda i: (i, 0)),
            pl.BlockSpec((H, EBLK), lambda i: (0, i)),
            pl.BlockSpec((EF, H), lambda i: (0, 0)),
            pl.BlockSpec((H, H), lambda i: (0, 0)),
            pl.BlockSpec((H, 1), lambda i: (0, 0)),
            pl.BlockSpec((3, H), lambda i: (0, 0)),
        ],
        out_specs=pl.BlockSpec((H, EBLK), lambda i: (0, i)),
        out_shape=jax.ShapeDtypeStruct((H, E), jnp.float32),
    )(e_raw, eta0, A1, lin_W1, c1.reshape(H, 1), bnm)


def _t2_body(s_ref, vr_ref, q2_ref, vs_ref, cnt_ref, w_ref, p2s_ref,
             ra_ref, rb_ref, rc_ref, rd_ref):
    s = s_ref[...]
    vs = vs_ref[...]
    cnt = cnt_ref[...]
    w = vr_ref[...] * s
    w_ref[...] = w
    p2s_ref[...] = q2_ref[...] * s
    ra_ref[...] = jnp.sum(cnt * vs, axis=1, keepdims=True)
    rb_ref[...] = jnp.sum(cnt * vs * vs, axis=1, keepdims=True)
    rc_ref[...] = jnp.sum(w, axis=1, keepdims=True)
    rd_ref[...] = jnp.sum(w * w, axis=1, keepdims=True)


def _t2(S_T, Vr_T, Q2_T, Vs_T, cnt):
    return pl.pallas_call(
        _t2_body,
        out_shape=[
            jax.ShapeDtypeStruct((H, N), jnp.float32),
            jax.ShapeDtypeStruct((H, N), jnp.float32),
            jax.ShapeDtypeStruct((H, 1), jnp.float32),
            jax.ShapeDtypeStruct((H, 1), jnp.float32),
            jax.ShapeDtypeStruct((H, 1), jnp.float32),
            jax.ShapeDtypeStruct((H, 1), jnp.float32),
        ],
    )(S_T, Vr_T, Q2_T, Vs_T, cnt.reshape(1, N))


def _t3_body(ht_ref, pt_ref, vs_ref, m_ref, q1_ref, mp_ref, bnm_ref,
             upd_ref, mm_ref, hto_ref, pto_ref):
    mu = bnm_ref[0:1, :].reshape(H, 1)
    sc = bnm_ref[1:2, :].reshape(H, 1)
    bb = bnm_ref[2:3, :].reshape(H, 1)
    upd = upd_ref[...]
    mm = mm_ref[...]
    tmp = vs_ref[...] + mm * m_ref[...]
    hto_ref[...] = ht_ref[...] + upd * jnp.maximum((tmp - mu) * sc + bb, 0.0)
    pto_ref[...] = pt_ref[...] + upd * jnp.tanh(q1_ref[...] + mm * mp_ref[...])


def _t3(ht, pt, Vs_T, M_T, Q1_T, Mp_T, bnm, upd, mm):
    return pl.pallas_call(
        _t3_body,
        out_shape=[jax.ShapeDtypeStruct((H, N), jnp.float32)] * 2,
    )(ht, pt, Vs_T, M_T, Q1_T, Mp_T, bnm, upd.reshape(1, N), mm.reshape(1, N))


def _t4a_body(ht_ref, pt_ref, b_ref, hagg_ref, pagg_ref):
    b = b_ref[...].reshape(N, 1)
    onehot = (lax.broadcasted_iota(jnp.int32, (N, G), 1) == b).astype(jnp.float32)
    hagg_ref[...] = lax.dot_general(ht_ref[...], onehot, (((1,), (0,)), ((), ())),
                                    preferred_element_type=jnp.float32)
    pagg_ref[...] = lax.dot_general(pt_ref[...], onehot, (((1,), (0,)), ((), ())),
                                    preferred_element_type=jnp.float32)


def _t4a(ht, pt, batch):
    return pl.pallas_call(
        _t4a_body,
        out_shape=[jax.ShapeDtypeStruct((H, G), jnp.float32)] * 2,
    )(ht, pt, batch.reshape(1, N))


def _t4b_body(ha_ref, pa_ref, r1a_ref, r1b_ref, r1bias_ref, r2w_ref, r2b_ref,
              r3w_ref, r3b_ref, out_ref):
    x1 = _dgT(r1a_ref[...], ha_ref[...]) + _dgT(r1b_ref[...], pa_ref[...]) + r1bias_ref[...]
    x1 = jnp.maximum(x1, 0.0)
    x2 = jnp.maximum(_dgT(r2w_ref[...], x1) + r2b_ref[...], 0.0)
    out_ref[...] = _dgT(r3w_ref[...], x2) + r3b_ref[...]


def _t4b(hagg, pagg, r1_W, r1_b, r2_W, r2_b, r3_W, r3_b):
    return pl.pallas_call(
        _t4b_body,
        out_shape=jax.ShapeDtypeStruct((1, G), jnp.float32),
    )(hagg, pagg, r1_W[:H], r1_W[H:], r1_b.reshape(H, 1), r2_W,
      r2_b.reshape(H // 2, 1), r3_W, r3_b.reshape(1, 1))


def _edge_pass(U_T, ete_T, send, rec):
    U = U_T.T
    eta = jax.nn.sigmoid(U[send] + U[rec] + ete_T.T)
    r = jnp.sum(eta, axis=1, keepdims=True)
    eta_new = eta / r
    S = jnp.zeros((N, H), jnp.float32).at[rec].add(eta_new)
    cnt = jnp.zeros((N,), jnp.float32).at[send].add(1.0)
    jmax = jnp.full((N,), -1, jnp.int32).at[send].max(
        jnp.arange(E, dtype=jnp.int32))
    sum_eta = jnp.sum(eta, axis=0)
    sumsq_eta = jnp.sum(eta * eta, axis=0)
    return S.T, cnt, jmax, sum_eta, sumsq_eta, eta.T


def _node_gather(Vs_T, W_T, P2S_T, send, jc):
    X = jnp.sum(Vs_T[:, send[:N]] * W_T, axis=1)
    M_T = W_T[:, jc]
    Mp_T = P2S_T[:, jc]
    return X, M_T, Mp_T


def kernel(h, e, p, edge_index, batch, Wh, bh, We, be, Wp, bp, lin_W, lin_b,
           hps_W, hps_b, hpr_W, hpr_b, p1_W, p1_b, p2_W, p2_b, bn_w, bn_b,
           r1_W, r1_b, r2_W, r2_b, r3_W, r3_b):
    send = edge_index[0]
    rec = edge_index[1]
    ht, pt = _t0(h, Wh, bh, p, Wp, bp)

    eta0 = None
    bnm_e0 = None
    for l in range(L):
        U_T, Vs_T, Vr_T, Q1_T, Q2_T = _t1(
            ht, pt, lin_W[l], lin_b[l], hps_W[l], hps_b[l], hpr_W[l], hpr_b[l],
            p1_W[l], p1_b[l], p2_W[l], p2_b[l])
        if l == 0:
            A0 = We @ lin_W[0]
            c0 = be @ lin_W[0] + lin_b[0]
            ete_T = _te0(e, A0, c0)
        else:
            A1 = We @ lin_W[1]
            c1 = be @ lin_W[1] + lin_b[1]
            ete_T = _te1(e, eta0, A1, lin_W[1], c1, bnm_e0)

        S_T, cnt, jmax, sum_eta, sumsq_eta, eta = _edge_pass(U_T, ete_T, send, rec)
        if l == 0:
            eta0 = eta

        mu_e = sum_eta / E
        var_e = sumsq_eta / E - mu_e * mu_e
        inv_e = bn_w[l] / jnp.sqrt(var_e + 1e-5)
        if l == 0:
            bnm_e0 = jnp.stack([mu_e, inv_e, bn_b[l]], axis=0)

        W_T, P2S_T, RA, RB, RC, RD = _t2(S_T, Vr_T, Q2_T, Vs_T, cnt)
        jc = jnp.clip(jmax, 0, N - 1)
        X, M_T, Mp_T = _node_gather(Vs_T, W_T, P2S_T, send, jc)

        sum_tmp = RA[:, 0] + RC[:, 0]
        sumsq_tmp = RB[:, 0] + 2.0 * X + RD[:, 0]
        mu_h = sum_tmp / E
        var_h = sumsq_tmp / E - mu_h * mu_h
        inv_h = bn_w[l] / jnp.sqrt(var_h + 1e-5)
        bnm_h = jnp.stack([mu_h, inv_h, bn_b[l]], axis=0)

        upd = (jmax >= 0).astype(jnp.float32)
        mm = ((jmax >= 0) & (jmax < N)).astype(jnp.float32)
        ht, pt = _t3(ht, pt, Vs_T, M_T, Q1_T, Mp_T, bnm_h, upd, mm)

    hagg, pagg = _t4a(ht, pt, batch)
    out = _t4b(hagg, pagg, r1_W, r1_b, r2_W, r2_b, r3_W, r3_b)
    return out.reshape(G)


# speedup vs baseline: 3.0111x; 1.9845x over previous
import jax
import jax.numpy as jnp
from jax import lax
from jax.experimental import pallas as pl
from jax.experimental.pallas import tpu as pltpu, tpu_sc as plsc

N = 10000
E = 160000
FEAT = 128
POS = 16
EF = 16
H = 64
L = 2
G = 128

EBLK = 6400
NSUB = 16
NCORE = 2
FPT = H // NSUB
BLK = 640
EHALF = E // NCORE
NBLKS = EHALF // BLK
NV = BLK // 16


def _dgT(w, x):
    return lax.dot_general(w, x, (((0,), (0,)), ((), ())),
                           preferred_element_type=jnp.float32)


def _store3(ref, val):
    for i in range(NSUB):
        ref[i] = val[FPT * i:FPT * (i + 1), :]


def _t0_body(h_ref, wh_ref, bh_ref, p_ref, wp_ref, bp_ref, ht_ref, pt_ref):
    h = h_ref[...]
    ht_ref[...] = lax.dot_general(wh_ref[...], h, (((0,), (1,)), ((), ())),
                                  preferred_element_type=jnp.float32) + bh_ref[...]
    p = p_ref[...]
    pt_ref[...] = lax.dot_general(wp_ref[...], p, (((0,), (1,)), ((), ())),
                                  preferred_element_type=jnp.float32) + bp_ref[...]


def _t0(h, Wh, bh, p, Wp, bp):
    return pl.pallas_call(
        _t0_body,
        out_shape=[
            jax.ShapeDtypeStruct((H, N), jnp.float32),
            jax.ShapeDtypeStruct((H, N), jnp.float32),
        ],
    )(h, Wh, bh.reshape(H, 1), p, Wp, bp.reshape(H, 1))


def _t1_body(ht_ref, pt_ref, linw_ref, linb_ref, hpsa_ref, hpsb_ref, hpsbias_ref,
             hpra_ref, hprb_ref, hprbias_ref, p1w_ref, p1b_ref, p2w_ref, p2b_ref,
             u_ref, vs_ref, vr_ref, q1_ref, q2_ref):
    ht = ht_ref[...]
    pt = pt_ref[...]
    _store3(u_ref, _dgT(linw_ref[...], ht) + linb_ref[...])
    _store3(vs_ref, _dgT(hpsa_ref[...], ht) + _dgT(hpsb_ref[...], pt) + hpsbias_ref[...])
    _store3(vr_ref, _dgT(hpra_ref[...], ht) + _dgT(hprb_ref[...], pt) + hprbias_ref[...])
    q1_ref[...] = _dgT(p1w_ref[...], pt) + p1b_ref[...]
    _store3(q2_ref, _dgT(p2w_ref[...], pt) + p2b_ref[...])


def _t1(ht, pt, lin_W, lin_b, hps_W, hps_b, hpr_W, hpr_b, p1_W, p1_b, p2_W, p2_b):
    o3 = jax.ShapeDtypeStruct((NSUB, FPT, N), jnp.float32)
    o2 = jax.ShapeDtypeStruct((H, N), jnp.float32)
    return pl.pallas_call(
        _t1_body,
        out_shape=[o3, o3, o3, o2, o3],
    )(ht, pt, lin_W, lin_b.reshape(H, 1),
      hps_W[:H], hps_W[H:], hps_b.reshape(H, 1),
      hpr_W[:H], hpr_W[H:], hpr_b.reshape(H, 1),
      p1_W, p1_b.reshape(H, 1), p2_W, p2_b.reshape(H, 1))


def _te0_body(e_ref, a_ref, c_ref, out_ref):
    r = lax.dot_general(a_ref[...], e_ref[...], (((0,), (1,)), ((), ())),
                        preferred_element_type=jnp.float32) + c_ref[...]
    _store3(out_ref, r)


def _te0(e_raw, A0, c0):
    grid = (E // EBLK,)
    return pl.pallas_call(
        _te0_body,
        grid=grid,
        in_specs=[
            pl.BlockSpec((EBLK, EF), lambda i: (i, 0)),
            pl.BlockSpec((EF, H), lambda i: (0, 0)),
            pl.BlockSpec((H, 1), lambda i: (0, 0)),
        ],
        out_specs=pl.BlockSpec((NSUB, FPT, EBLK), lambda i: (0, 0, i)),
        out_shape=jax.ShapeDtypeStruct((NSUB, FPT, E), jnp.float32),
    )(e_raw, A0, c0.reshape(H, 1))


def _te1_body(e_ref, eta_ref, a_ref, lw_ref, c_ref, bnm_ref, out_ref):
    eta = jnp.concatenate([eta_ref[i] for i in range(NSUB)], axis=0)
    mu = bnm_ref[0:1, :].reshape(H, 1)
    sc = bnm_ref[1:2, :].reshape(H, 1)
    bb = bnm_ref[2:3, :].reshape(H, 1)
    r = jnp.maximum((eta - mu) * sc + bb, 0.0)
    out = (lax.dot_general(a_ref[...], e_ref[...], (((0,), (1,)), ((), ())),
                           preferred_element_type=jnp.float32)
           + _dgT(lw_ref[...], r) + c_ref[...])
    _store3(out_ref, out)


def _te1(e_raw, eta0, A1, lin_W1, c1, bnm):
    grid = (E // EBLK,)
    return pl.pallas_call(
        _te1_body,
        grid=grid,
        in_specs=[
            pl.BlockSpec((EBLK, EF), lambda i: (i, 0)),
            pl.BlockSpec((NSUB, FPT, EBLK), lambda i: (0, 0, i)),
            pl.BlockSpec((EF, H), lambda i: (0, 0)),
            pl.BlockSpec((H, H), lambda i: (0, 0)),
            pl.BlockSpec((H, 1), lambda i: (0, 0)),
            pl.BlockSpec((3, H), lambda i: (0, 0)),
        ],
        out_specs=pl.BlockSpec((NSUB, FPT, EBLK), lambda i: (0, 0, i)),
        out_shape=jax.ShapeDtypeStruct((NSUB, FPT, E), jnp.float32),
    )(e_raw, eta0, A1, lin_W1, c1.reshape(H, 1), bnm)


def _t2_body(s_ref, vr_ref, q2_ref, vs_ref, cnt_ref, jm_ref, w_ref, p2s_ref,
             ra_ref, rb_ref, rc_ref, rd_ref, jc_ref, upd_ref, mm_ref):
    cnt2 = cnt_ref[0] + cnt_ref[1]
    cnt = jnp.sum(cnt2, axis=0, keepdims=True)
    jm2 = jnp.maximum(jm_ref[0], jm_ref[1])
    jmax = jnp.max(jm2, axis=0, keepdims=True)
    for i in range(NSUB):
        s = s_ref[0, i] + s_ref[1, i]
        vs = vs_ref[i]
        w = vr_ref[i] * s
        w_ref[i] = w
        p2s_ref[i] = q2_ref[i] * s
        ra_ref[i] = jnp.sum(cnt * vs, axis=1)
        rb_ref[i] = jnp.sum(cnt * vs * vs, axis=1)
        rc_ref[i] = jnp.sum(w, axis=1)
        rd_ref[i] = jnp.sum(w * w, axis=1)
    jc_ref[...] = jnp.clip(jmax, 0, N - 1)
    upd_ref[...] = (jmax >= 0).astype(jnp.float32)
    mm_ref[...] = ((jmax >= 0) & (jmax < N)).astype(jnp.float32)


def _t2(S_part, Vr3, Q2_3, Vs3, cnt_part, jmax_part):
    o3 = jax.ShapeDtypeStruct((NSUB, FPT, N), jnp.float32)
    r3 = jax.ShapeDtypeStruct((NSUB, FPT), jnp.float32)
    return pl.pallas_call(
        _t2_body,
        out_shape=[o3, o3, r3, r3, r3, r3,
                   jax.ShapeDtypeStruct((1, N), jnp.int32),
                   jax.ShapeDtypeStruct((1, N), jnp.float32),
                   jax.ShapeDtypeStruct((1, N), jnp.float32)],
    )(S_part, Vr3, Q2_3, Vs3, cnt_part, jmax_part)


def _t3_body(ht_ref, pt_ref, vs_ref, m_ref, q1_ref, mp_ref, bnm_ref,
             upd_ref, mm_ref, hto_ref, pto_ref):
    upd = upd_ref[...]
    mm = mm_ref[...]
    q1 = q1_ref[...]
    for i in range(NSUB):
        sl = slice(FPT * i, FPT * (i + 1))
        mu = bnm_ref[0:1, sl].reshape(FPT, 1)
        sc = bnm_ref[1:2, sl].reshape(FPT, 1)
        bb = bnm_ref[2:3, sl].reshape(FPT, 1)
        tmp = vs_ref[i] + mm * m_ref[i]
        hto_ref[sl, :] = ht_ref[sl, :] + upd * jnp.maximum((tmp - mu) * sc + bb, 0.0)
        pto_ref[sl, :] = pt_ref[sl, :] + upd * jnp.tanh(q1[sl, :] + mm * mp_ref[i])


def _t3(ht, pt, Vs3, M3, Q1_T, Mp3, bnm, upd, mm):
    return pl.pallas_call(
        _t3_body,
        out_shape=[jax.ShapeDtypeStruct((H, N), jnp.float32)] * 2,
    )(ht, pt, Vs3, M3, Q1_T, Mp3, bnm, upd, mm)


def _t4a_body(ht_ref, pt_ref, b_ref, hagg_ref, pagg_ref):
    b = b_ref[...].reshape(N, 1)
    onehot = (lax.broadcasted_iota(jnp.int32, (N, G), 1) == b).astype(jnp.float32)
    hagg_ref[...] = lax.dot_general(ht_ref[...], onehot, (((1,), (0,)), ((), ())),
                                    preferred_element_type=jnp.float32)
    pagg_ref[...] = lax.dot_general(pt_ref[...], onehot, (((1,), (0,)), ((), ())),
                                    preferred_element_type=jnp.float32)


def _t4a(ht, pt, batch):
    return pl.pallas_call(
        _t4a_body,
        out_shape=[jax.ShapeDtypeStruct((H, G), jnp.float32)] * 2,
    )(ht, pt, batch.reshape(1, N))


def _t4b_body(ha_ref, pa_ref, r1a_ref, r1b_ref, r1bias_ref, r2w_ref, r2b_ref,
              r3w_ref, r3b_ref, out_ref):
    x1 = _dgT(r1a_ref[...], ha_ref[...]) + _dgT(r1b_ref[...], pa_ref[...]) + r1bias_ref[...]
    x1 = jnp.maximum(x1, 0.0)
    x2 = jnp.maximum(_dgT(r2w_ref[...], x1) + r2b_ref[...], 0.0)
    out_ref[...] = _dgT(r3w_ref[...], x2) + r3b_ref[...]


def _t4b(hagg, pagg, r1_W, r1_b, r2_W, r2_b, r3_W, r3_b):
    return pl.pallas_call(
        _t4b_body,
        out_shape=jax.ShapeDtypeStruct((1, G), jnp.float32),
    )(hagg, pagg, r1_W[:H], r1_W[H:], r1_b.reshape(H, 1), r2_W,
      r2_b.reshape(H // 2, 1), r3_W, r3_b.reshape(1, 1))


def _k1_body(emit_eta, u_hbm, ete_hbm, send_hbm, rec_hbm,
             s_out, cnt_out, jmax_out, asum_out, asq_out, eta_out,
             u_tbl, s_tbl, cnt_tbl, jmax_tbl, send_buf, rec_buf,
             ete_buf, eta_buf, rsum, red_in, acc_buf, sh_part, sh_tot):
    co = lax.axis_index("c")
    s = lax.axis_index("s")
    ebase = co * EHALF

    pltpu.sync_copy(u_hbm.at[s], u_tbl)

    def zloop(i, _):
        dv = pl.ds(i * 16, 16)
        zf = jnp.zeros((16,), jnp.float32)
        for f in range(FPT):
            s_tbl[f, dv] = zf
        cnt_tbl[dv] = zf
        jmax_tbl[dv] = jnp.full((16,), -1, jnp.int32)
        return 0

    lax.fori_loop(0, N // 16, zloop, 0)

    def block_body(b, accs):
        a_s, a_q = accs
        off = ebase + b * BLK
        pltpu.sync_copy(send_hbm.at[pl.ds(off, BLK)], send_buf)
        pltpu.sync_copy(rec_hbm.at[pl.ds(off, BLK)], rec_buf)
        pltpu.sync_copy(ete_hbm.at[s, :, pl.ds(off, BLK)], ete_buf)

        def vloop(v, carry):
            ca_s, ca_q = carry
            dv = pl.ds(v * 16, 16)
            s16 = send_buf[dv]
            r16 = rec_buf[dv]
            rs = jnp.zeros((16,), jnp.float32)
            na_s, na_q = [], []
            for f in range(FPT):
                fv = jnp.full((16,), f, jnp.int32)
                us = plsc.load_gather(u_tbl, [fv, s16])
                ur = plsc.load_gather(u_tbl, [fv, r16])
                z = us + ur + ete_buf[f, dv]
                eta = 1.0 / (1.0 + jnp.exp(-z))
                eta_buf[f, dv] = eta
                rs = rs + eta
                na_s.append(ca_s[f] + eta)
                na_q.append(ca_q[f] + eta * eta)
            rsum[dv] = rs
            return (tuple(na_s), tuple(na_q))

        a_s, a_q = lax.fori_loop(0, NV, vloop, (a_s, a_q))

        pltpu.sync_copy(rsum, sh_part.at[s])
        plsc.subcore_barrier()

        @pl.when(s < BLK // 128)
        def _():
            pltpu.sync_copy(sh_part.at[:, pl.ds(s * 128, 128)], red_in)

            def redloop(v, _):
                dv = pl.ds(v * 16, 16)
                t = red_in[0, dv]
                for r in range(1, NSUB):
                    t = t + red_in[r, dv]
                rsum[pl.ds(s * 128 + v * 16, 16)] = t
                return 0

            lax.fori_loop(0, 128 // 16, redloop, 0)
            pltpu.sync_copy(rsum.at[pl.ds(s * 128, 128)],
                            sh_tot.at[pl.ds(s * 128, 128)])

        plsc.subcore_barrier()
        pltpu.sync_copy(sh_tot, rsum)

        def vloop2(v, _):
            dv = pl.ds(v * 16, 16)
            r16 = rec_buf[dv]
            rinv = 1.0 / rsum[dv]
            for f in range(FPT):
                fv = jnp.full((16,), f, jnp.int32)
                en = eta_buf[f, dv] * rinv
                plsc.addupdate_scatter(s_tbl, [fv, r16], en)
            return 0

        lax.fori_loop(0, NV, vloop2, 0)
        if emit_eta:
            pltpu.sync_copy(eta_buf, eta_out.at[s, :, pl.ds(off, BLK)])

        @pl.when(s >= NSUB - BLK // 128)
        def _():
            cbase = (s - (NSUB - BLK // 128)) * 128

            def jloop(v, _):
                col = cbase + v * 16
                dv = pl.ds(col, 16)
                j16 = off + col + lax.iota(jnp.int32, 16)
                idx = send_buf[dv]
                plsc.addupdate_scatter(cnt_tbl, [idx], jnp.ones((16,), jnp.float32))
                plsc.store_scatter(jmax_tbl, [idx], j16)
                g = plsc.load_gather(jmax_tbl, [idx])

                def wbody(active):
                    g2 = plsc.load_gather(jmax_tbl, [idx])
                    m = j16 > g2
                    plsc.store_scatter(jmax_tbl, [idx], j16, mask=m)
                    g3 = plsc.load_gather(jmax_tbl, [idx])
                    return jnp.any(j16 > g3)

                lax.while_loop(lambda a: a, wbody, jnp.any(j16 > g))
                return 0

            lax.fori_loop(0, 128 // 16, jloop, 0)

        return (a_s, a_q)

    zero16 = jnp.zeros((16,), jnp.float32)
    init = (tuple(zero16 for _ in range(FPT)), tuple(zero16 for _ in range(FPT)))
    a_s, a_q = lax.fori_loop(0, NBLKS, block_body, init)

    def zacc(i, _):
        zf = jnp.zeros((16,), jnp.float32)
        for f in range(FPT):
            acc_buf[f, pl.ds(i * 16, 16)] = zf
        return 0

    lax.fori_loop(0, 8, zacc, 0)
    for f in range(FPT):
        acc_buf[f, pl.ds(0, 16)] = a_s[f]
    pltpu.sync_copy(acc_buf, asum_out.at[co, s])
    for f in range(FPT):
        acc_buf[f, pl.ds(0, 16)] = a_q[f]
    pltpu.sync_copy(acc_buf, asq_out.at[co, s])
    pltpu.sync_copy(s_tbl, s_out.at[co, s])
    pltpu.sync_copy(cnt_tbl, cnt_out.at[co, s])
    pltpu.sync_copy(jmax_tbl, jmax_out.at[co, s])


def _make_k1(emit_eta):
    mesh = plsc.VectorSubcoreMesh(core_axis_name="c", subcore_axis_name="s",
                                  num_cores=NCORE, num_subcores=NSUB)
    outs = [
        jax.ShapeDtypeStruct((NCORE, NSUB, FPT, N), jnp.float32),
        jax.ShapeDtypeStruct((NCORE, NSUB, N), jnp.float32),
        jax.ShapeDtypeStruct((NCORE, NSUB, N), jnp.int32),
        jax.ShapeDtypeStruct((NCORE, NSUB, FPT, 128), jnp.float32),
        jax.ShapeDtypeStruct((NCORE, NSUB, FPT, 128), jnp.float32),
        jax.ShapeDtypeStruct((NSUB, FPT, E), jnp.float32),
    ]
    if not emit_eta:
        outs = outs[:-1]
    scratch = [
        pltpu.VMEM((FPT, N), jnp.float32),
        pltpu.VMEM((FPT, N), jnp.float32),
        pltpu.VMEM((N,), jnp.float32),
        pltpu.VMEM((N,), jnp.int32),
        pltpu.VMEM((BLK,), jnp.int32),
        pltpu.VMEM((BLK,), jnp.int32),
        pltpu.VMEM((FPT, BLK), jnp.float32),
        pltpu.VMEM((FPT, BLK), jnp.float32),
        pltpu.VMEM((BLK,), jnp.float32),
        pltpu.VMEM((NSUB, 128), jnp.float32),
        pltpu.VMEM((FPT, 128), jnp.float32),
        pltpu.VMEM_SHARED((NSUB, BLK), jnp.float32),
        pltpu.VMEM_SHARED((BLK,), jnp.float32),
    ]

    def body(*refs):
        if emit_eta:
            _k1_body(True, *refs)
        else:
            ins, rest = refs[:4], refs[4:]
            _k1_body(False, *ins, *rest[:5], None, *rest[5:])

    return pl.kernel(body, out_type=outs, mesh=mesh, scratch_types=scratch,
                     compiler_params=pltpu.CompilerParams(needs_layout_passes=False))


_k1_cache = {}


def _get_k1(emit_eta):
    if emit_eta not in _k1_cache:
        _k1_cache[emit_eta] = _make_k1(emit_eta)
    return _k1_cache[emit_eta]


def _k2_body(vs_hbm, w_hbm, p2s_hbm, send_hbm, jc_hbm,
             m_out, mp_out, x_out,
             tblA, tblB, idx1, xstage):
    co = lax.axis_index("c")
    s = lax.axis_index("s")

    @pl.when(co == 0)
    def _():
        pltpu.sync_copy(vs_hbm.at[s], tblA)
        pltpu.sync_copy(w_hbm.at[s], tblB)
        pltpu.sync_copy(send_hbm.at[pl.ds(0, N)], idx1)

        def xloop(v, carry):
            dv = pl.ds(v * 16, 16)
            s16 = idx1[dv]
            nc = []
            for f in range(FPT):
                fv = jnp.full((16,), f, jnp.int32)
                g = plsc.load_gather(tblA, [fv, s16])
                nc.append(carry[f] + g * tblB[f, dv])
            return tuple(nc)

        z16 = jnp.zeros((16,), jnp.float32)
        xs = lax.fori_loop(0, N // 16, xloop, tuple(z16 for _ in range(FPT)))

        def zx(i, _):
            for f in range(FPT):
                xstage[f, pl.ds(i * 16, 16)] = jnp.zeros((16,), jnp.float32)
            return 0

        lax.fori_loop(0, 8, zx, 0)
        for f in range(FPT):
            xstage[f, pl.ds(0, 16)] = xs[f]
        pltpu.sync_copy(xstage, x_out.at[s])

        pltpu.sync_copy(p2s_hbm.at[s], tblA)
        pltpu.sync_copy(jc_hbm.at[pl.ds(0, N)], idx1)

        def mploop(v, _):
            dv = pl.ds(v * 16, 16)
            j16 = idx1[dv]
            for f in range(FPT):
                fv = jnp.full((16,), f, jnp.int32)
                tblB[f, dv] = plsc.load_gather(tblA, [fv, j16])
            return 0

        lax.fori_loop(0, N // 16, mploop, 0)
        pltpu.sync_copy(tblB, mp_out.at[s])

    @pl.when(co == 1)
    def _():
        pltpu.sync_copy(w_hbm.at[s], tblA)
        pltpu.sync_copy(jc_hbm.at[pl.ds(0, N)], idx1)

        def mloop(v, _):
            dv = pl.ds(v * 16, 16)
            j16 = idx1[dv]
            for f in range(FPT):
                fv = jnp.full((16,), f, jnp.int32)
                tblB[f, dv] = plsc.load_gather(tblA, [fv, j16])
            return 0

        lax.fori_loop(0, N // 16, mloop, 0)
        pltpu.sync_copy(tblB, m_out.at[s])


_k2_cache = {}


def _get_k2():
    if "k" not in _k2_cache:
        mesh = plsc.VectorSubcoreMesh(core_axis_name="c", subcore_axis_name="s",
                                      num_cores=NCORE, num_subcores=NSUB)
        _k2_cache["k"] = pl.kernel(
            _k2_body,
            out_type=[
                jax.ShapeDtypeStruct((NSUB, FPT, N), jnp.float32),
                jax.ShapeDtypeStruct((NSUB, FPT, N), jnp.float32),
                jax.ShapeDtypeStruct((NSUB, FPT, 128), jnp.float32),
            ],
            mesh=mesh,
            scratch_types=[
                pltpu.VMEM((FPT, N), jnp.float32),
                pltpu.VMEM((FPT, N), jnp.float32),
                pltpu.VMEM((N,), jnp.int32),
                pltpu.VMEM((FPT, 128), jnp.float32),
            ],
            compiler_params=pltpu.CompilerParams(needs_layout_passes=False))
    return _k2_cache["k"]


def _edge_pass(U3, ete3, send, rec, emit_eta):
    if emit_eta:
        S_part, cnt_part, jmax_part, asum, asq, eta = _get_k1(True)(U3, ete3, send, rec)
    else:
        S_part, cnt_part, jmax_part, asum, asq = _get_k1(False)(U3, ete3, send, rec)
        eta = None
    sum_eta = jnp.sum(asum, axis=(0, 3)).reshape(H)
    sumsq_eta = jnp.sum(asq, axis=(0, 3)).reshape(H)
    return S_part, cnt_part, jmax_part, sum_eta, sumsq_eta, eta


def _node_gather(Vs3, W3, P2S3, send, jc):
    M3, Mp3, X_part = _get_k2()(Vs3, W3, P2S3, send, jc)
    return jnp.sum(X_part, axis=2).reshape(H), M3, Mp3


def kernel(h, e, p, edge_index, batch, Wh, bh, We, be, Wp, bp, lin_W, lin_b,
           hps_W, hps_b, hpr_W, hpr_b, p1_W, p1_b, p2_W, p2_b, bn_w, bn_b,
           r1_W, r1_b, r2_W, r2_b, r3_W, r3_b):
    send = edge_index[0]
    rec = edge_index[1]
    ht, pt = _t0(h, Wh, bh, p, Wp, bp)

    eta0 = None
    bnm_e0 = None
    for l in range(L):
        U3, Vs3, Vr3, Q1_T, Q2_3 = _t1(
            ht, pt, lin_W[l], lin_b[l], hps_W[l], hps_b[l], hpr_W[l], hpr_b[l],
            p1_W[l], p1_b[l], p2_W[l], p2_b[l])
        if l == 0:
            A0 = We @ lin_W[0]
            c0 = be @ lin_W[0] + lin_b[0]
            ete3 = _te0(e, A0, c0)
        else:
            A1 = We @ lin_W[1]
            c1 = be @ lin_W[1] + lin_b[1]
            ete3 = _te1(e, eta0, A1, lin_W[1], c1, bnm_e0)

        (S_part, cnt_part, jmax_part, sum_eta, sumsq_eta,
         eta) = _edge_pass(U3, ete3, send, rec, l == 0)
        if l == 0:
            eta0 = eta

        mu_e = sum_eta / E
        var_e = sumsq_eta / E - mu_e * mu_e
        inv_e = bn_w[l] / jnp.sqrt(var_e + 1e-5)
        if l == 0:
            bnm_e0 = jnp.stack([mu_e, inv_e, bn_b[l]], axis=0)

        W3, P2S3, RA, RB, RC, RD, jc, upd, mm = _t2(
            S_part, Vr3, Q2_3, Vs3, cnt_part, jmax_part)
        X, M3, Mp3 = _node_gather(Vs3, W3, P2S3, send, jc.reshape(N))

        sum_tmp = RA.reshape(H) + RC.reshape(H)
        sumsq_tmp = RB.reshape(H) + 2.0 * X + RD.reshape(H)
        mu_h = sum_tmp / E
        var_h = sumsq_tmp / E - mu_h * mu_h
        inv_h = bn_w[l] / jnp.sqrt(var_h + 1e-5)
        bnm_h = jnp.stack([mu_h, inv_h, bn_b[l]], axis=0)

        ht, pt = _t3(ht, pt, Vs3, M3, Q1_T, Mp3, bnm_h, upd, mm)

    hagg, pagg = _t4a(ht, pt, batch)
    out = _t4b(hagg, pagg, r1_W, r1_b, r2_W, r2_b, r3_W, r3_b)
    return out.reshape(G)


# speedup vs baseline: 3.8714x; 1.2857x over previous
import jax
import jax.numpy as jnp
from jax import lax
from jax.experimental import pallas as pl
from jax.experimental.pallas import tpu as pltpu, tpu_sc as plsc

N = 10000
E = 160000
FEAT = 128
POS = 16
EF = 16
H = 64
L = 2
G = 128

EBLK = 6400
NSUB = 16
NCORE = 2
FPT = H // NSUB
BLK = 3200
EHALF = E // NCORE
NBLKS = EHALF // BLK
NV = BLK // 16


def _dgT(w, x):
    return lax.dot_general(w, x, (((0,), (0,)), ((), ())),
                           preferred_element_type=jnp.float32)


def _store3(ref, val):
    for i in range(NSUB):
        ref[i] = val[FPT * i:FPT * (i + 1), :]


def _t0_body(h_ref, wh_ref, bh_ref, p_ref, wp_ref, bp_ref, ht_ref, pt_ref):
    h = h_ref[...]
    ht_ref[...] = lax.dot_general(wh_ref[...], h, (((0,), (1,)), ((), ())),
                                  preferred_element_type=jnp.float32) + bh_ref[...]
    p = p_ref[...]
    pt_ref[...] = lax.dot_general(wp_ref[...], p, (((0,), (1,)), ((), ())),
                                  preferred_element_type=jnp.float32) + bp_ref[...]


def _t0(h, Wh, bh, p, Wp, bp):
    return pl.pallas_call(
        _t0_body,
        out_shape=[
            jax.ShapeDtypeStruct((H, N), jnp.float32),
            jax.ShapeDtypeStruct((H, N), jnp.float32),
        ],
    )(h, Wh, bh.reshape(H, 1), p, Wp, bp.reshape(H, 1))


def _t1_body(ht_ref, pt_ref, linw_ref, linb_ref, hps_ref, hpsbias_ref,
             hpr_ref, hprbias_ref, p1w_ref, p1b_ref, p2w_ref, p2b_ref,
             u_ref, vs_ref, vr_ref, q1_ref, q2_ref):
    ht = ht_ref[...]
    pt = pt_ref[...]
    hp = jnp.concatenate([ht, pt], axis=0)
    _store3(u_ref, _dgT(linw_ref[...], ht) + linb_ref[...])
    _store3(vs_ref, _dgT(hps_ref[...], hp) + hpsbias_ref[...])
    _store3(vr_ref, _dgT(hpr_ref[...], hp) + hprbias_ref[...])
    q1_ref[...] = _dgT(p1w_ref[...], pt) + p1b_ref[...]
    _store3(q2_ref, _dgT(p2w_ref[...], pt) + p2b_ref[...])


def _t1(ht, pt, lin_W, lin_b, hps_W, hps_b, hpr_W, hpr_b, p1_W, p1_b, p2_W, p2_b):
    o3 = jax.ShapeDtypeStruct((NSUB, FPT, N), jnp.float32)
    o2 = jax.ShapeDtypeStruct((H, N), jnp.float32)
    return pl.pallas_call(
        _t1_body,
        out_shape=[o3, o3, o3, o2, o3],
    )(ht, pt, lin_W, lin_b.reshape(H, 1),
      hps_W, hps_b.reshape(H, 1),
      hpr_W, hpr_b.reshape(H, 1),
      p1_W, p1_b.reshape(H, 1), p2_W, p2_b.reshape(H, 1))


def _te0_body(e_ref, we_ref, be_ref, lw_ref, lb_ref, out_ref):
    e0 = lax.dot_general(we_ref[...], e_ref[...], (((0,), (1,)), ((), ())),
                         preferred_element_type=jnp.float32) + be_ref[...]
    r = _dgT(lw_ref[...], e0) + lb_ref[...]
    _store3(out_ref, r)


def _te0(e_raw, We, be, lin_W0, lin_b0):
    grid = (E // EBLK,)
    return pl.pallas_call(
        _te0_body,
        grid=grid,
        in_specs=[
            pl.BlockSpec((EBLK, EF), lambda i: (i, 0)),
            pl.BlockSpec((EF, H), lambda i: (0, 0)),
            pl.BlockSpec((H, 1), lambda i: (0, 0)),
            pl.BlockSpec((H, H), lambda i: (0, 0)),
            pl.BlockSpec((H, 1), lambda i: (0, 0)),
        ],
        out_specs=pl.BlockSpec((NSUB, FPT, EBLK), lambda i: (0, 0, i)),
        out_shape=jax.ShapeDtypeStruct((NSUB, FPT, E), jnp.float32),
    )(e_raw, We, be.reshape(H, 1), lin_W0, lin_b0.reshape(H, 1))


def _te1_body(e_ref, eta_ref, we_ref, be_ref, lw_ref, lb_ref, bnm_ref, out_ref):
    eta = jnp.concatenate([eta_ref[i] for i in range(NSUB)], axis=0)
    mu = bnm_ref[0:1, :].reshape(H, 1)
    sc = bnm_ref[1:2, :].reshape(H, 1)
    bb = bnm_ref[2:3, :].reshape(H, 1)
    r = jnp.maximum((eta - mu) * sc + bb, 0.0)
    e0 = lax.dot_general(we_ref[...], e_ref[...], (((0,), (1,)), ((), ())),
                         preferred_element_type=jnp.float32) + be_ref[...]
    e1 = e0 + r
    out = _dgT(lw_ref[...], e1) + lb_ref[...]
    _store3(out_ref, out)


def _te1(e_raw, eta0, We, be, lin_W1, lin_b1, bnm):
    grid = (E // EBLK,)
    return pl.pallas_call(
        _te1_body,
        grid=grid,
        in_specs=[
            pl.BlockSpec((EBLK, EF), lambda i: (i, 0)),
            pl.BlockSpec((NSUB, FPT, EBLK), lambda i: (0, 0, i)),
            pl.BlockSpec((EF, H), lambda i: (0, 0)),
            pl.BlockSpec((H, 1), lambda i: (0, 0)),
            pl.BlockSpec((H, H), lambda i: (0, 0)),
            pl.BlockSpec((H, 1), lambda i: (0, 0)),
            pl.BlockSpec((3, H), lambda i: (0, 0)),
        ],
        out_specs=pl.BlockSpec((NSUB, FPT, EBLK), lambda i: (0, 0, i)),
        out_shape=jax.ShapeDtypeStruct((NSUB, FPT, E), jnp.float32),
    )(e_raw, eta0, We, be.reshape(H, 1), lin_W1, lin_b1.reshape(H, 1), bnm)


def _t2_body(s_ref, vr_ref, q2_ref, vs_ref, cnt_ref, w_ref, p2s_ref,
             ra_ref, rb_ref, rc_ref, rd_ref, mu_ref):
    cnt = cnt_ref[...]
    for i in range(NSUB):
        s = s_ref[0, i] + s_ref[1, i]
        vs = vs_ref[i]
        w = vr_ref[i] * s
        w_ref[i] = w
        p2s_ref[i] = q2_ref[i] * s
        ra = jnp.sum(cnt * vs, axis=1)
        rc = jnp.sum(w, axis=1)
        mu = (ra + rc) * (1.0 / E)
        vsc = vs - mu.reshape(FPT, 1)
        ra_ref[i] = ra
        rb_ref[i] = jnp.sum(cnt * vsc * vsc, axis=1)
        rc_ref[i] = rc
        rd_ref[i] = jnp.sum(w * w, axis=1)
        mu_ref[i] = jnp.broadcast_to(mu.reshape(FPT, 1), (FPT, 128))


def _t2(S_part, Vr3, Q2_3, Vs3, cnt):
    o3 = jax.ShapeDtypeStruct((NSUB, FPT, N), jnp.float32)
    r3 = jax.ShapeDtypeStruct((NSUB, FPT), jnp.float32)
    return pl.pallas_call(
        _t2_body,
        out_shape=[o3, o3, r3, r3, r3, r3,
                   jax.ShapeDtypeStruct((NSUB, FPT, 128), jnp.float32)],
    )(S_part, Vr3, Q2_3, Vs3, cnt)


def _tj_body(cnt_ref, jm_ref, cnt_out, jc_out, upd_out, mm_out):
    cnt2 = cnt_ref[0] + cnt_ref[1]
    cnt_out[...] = jnp.sum(cnt2, axis=0, keepdims=True)
    jm2 = jnp.maximum(jm_ref[0], jm_ref[1])
    jmax = jnp.max(jm2, axis=0, keepdims=True)
    jc_out[...] = jnp.clip(jmax, 0, N - 1)
    upd_out[...] = (jmax >= 0).astype(jnp.float32)
    mm_out[...] = ((jmax >= 0) & (jmax < N)).astype(jnp.float32)


def _tj(cnt_part, jmax_part):
    return pl.pallas_call(
        _tj_body,
        out_shape=[jax.ShapeDtypeStruct((1, N), jnp.float32),
                   jax.ShapeDtypeStruct((1, N), jnp.int32),
                   jax.ShapeDtypeStruct((1, N), jnp.float32),
                   jax.ShapeDtypeStruct((1, N), jnp.float32)],
    )(cnt_part, jmax_part)


def _t3_body(ht_ref, pt_ref, vs_ref, m_ref, q1_ref, mp_ref, bnm_ref,
             upd_ref, mm_ref, hto_ref, pto_ref):
    upd = upd_ref[...]
    mm = mm_ref[...]
    q1 = q1_ref[...]
    for i in range(NSUB):
        sl = slice(FPT * i, FPT * (i + 1))
        mu = bnm_ref[0:1, sl].reshape(FPT, 1)
        sc = bnm_ref[1:2, sl].reshape(FPT, 1)
        bb = bnm_ref[2:3, sl].reshape(FPT, 1)
        tmp = vs_ref[i] + mm * m_ref[i]
        hto_ref[sl, :] = ht_ref[sl, :] + upd * jnp.maximum((tmp - mu) * sc + bb, 0.0)
        pto_ref[sl, :] = pt_ref[sl, :] + upd * jnp.tanh(q1[sl, :] + mm * mp_ref[i])


def _t3(ht, pt, Vs3, M3, Q1_T, Mp3, bnm, upd, mm):
    return pl.pallas_call(
        _t3_body,
        out_shape=[jax.ShapeDtypeStruct((H, N), jnp.float32)] * 2,
    )(ht, pt, Vs3, M3, Q1_T, Mp3, bnm, upd, mm)


def _t4a_body(ht_ref, pt_ref, b_ref, hagg_ref, pagg_ref):
    b = b_ref[...].reshape(N, 1)
    onehot = (lax.broadcasted_iota(jnp.int32, (N, G), 1) == b).astype(jnp.float32)
    hagg_ref[...] = lax.dot_general(ht_ref[...], onehot, (((1,), (0,)), ((), ())),
                                    preferred_element_type=jnp.float32,
                                    precision=lax.Precision.HIGHEST)
    pagg_ref[...] = lax.dot_general(pt_ref[...], onehot, (((1,), (0,)), ((), ())),
                                    preferred_element_type=jnp.float32,
                                    precision=lax.Precision.HIGHEST)


def _t4a(ht, pt, batch):
    return pl.pallas_call(
        _t4a_body,
        out_shape=[jax.ShapeDtypeStruct((H, G), jnp.float32)] * 2,
    )(ht, pt, batch.reshape(1, N))


def _t4b_body(ha_ref, pa_ref, r1w_ref, r1bias_ref, r2w_ref, r2b_ref,
              r3w_ref, r3b_ref, out_ref):
    hep = jnp.concatenate([ha_ref[...], pa_ref[...]], axis=0)
    x1 = _dgT(r1w_ref[...], hep) + r1bias_ref[...]
    x1 = jnp.maximum(x1, 0.0)
    x2 = jnp.maximum(_dgT(r2w_ref[...], x1) + r2b_ref[...], 0.0)
    out_ref[...] = _dgT(r3w_ref[...], x2) + r3b_ref[...]


def _t4b(hagg, pagg, r1_W, r1_b, r2_W, r2_b, r3_W, r3_b):
    return pl.pallas_call(
        _t4b_body,
        out_shape=jax.ShapeDtypeStruct((1, G), jnp.float32),
    )(hagg, pagg, r1_W, r1_b.reshape(H, 1), r2_W,
      r2_b.reshape(H // 2, 1), r3_W, r3_b.reshape(1, 1))


def _k1_body(emit_eta, u_hbm, ete_hbm, send_hbm, rec_hbm,
             s_out, asum_out, asq_out, eta_out,
             u_tbl, s_tbl, send_buf, rec_buf,
             ete_buf, eta_buf, rsum, red_in, acc_buf, sh_part, sh_tot):
    co = lax.axis_index("c")
    s = lax.axis_index("s")
    ebase = co * EHALF

    pltpu.sync_copy(u_hbm.at[s], u_tbl)

    def zloop(i, _):
        dv = pl.ds(i * 16, 16)
        zf = jnp.zeros((16,), jnp.float32)
        for f in range(FPT):
            s_tbl[f, dv] = zf
        return 0

    lax.fori_loop(0, N // 16, zloop, 0)

    def block_body(b, accs):
        a_s, a_q = accs
        off = ebase + b * BLK
        pltpu.sync_copy(send_hbm.at[pl.ds(off, BLK)], send_buf)
        pltpu.sync_copy(rec_hbm.at[pl.ds(off, BLK)], rec_buf)
        pltpu.sync_copy(ete_hbm.at[s, :, pl.ds(off, BLK)], ete_buf)

        def vloop(v, carry):
            ca_s, ca_q = carry
            dv = pl.ds(v * 16, 16)
            s16 = send_buf[dv]
            r16 = rec_buf[dv]
            rs = jnp.zeros((16,), jnp.float32)
            na_s, na_q = [], []
            for f in range(FPT):
                fv = jnp.full((16,), f, jnp.int32)
                us = plsc.load_gather(u_tbl, [fv, s16])
                ur = plsc.load_gather(u_tbl, [fv, r16])
                z = us + ur + ete_buf[f, dv]
                eta = 1.0 / (1.0 + jnp.exp(-z))
                eta_buf[f, dv] = eta
                rs = rs + eta
                ec = eta - 0.5
                na_s.append(ca_s[f] + ec)
                na_q.append(ca_q[f] + ec * ec)
            rsum[dv] = rs
            return (tuple(na_s), tuple(na_q))

        a_s, a_q = lax.fori_loop(0, NV, vloop, (a_s, a_q))

        pltpu.sync_copy(rsum, sh_part.at[s])
        plsc.subcore_barrier()

        for rr in range((BLK // 128 + NSUB - 1) // NSUB):
            sl = rr * NSUB + s

            @pl.when(sl < BLK // 128)
            def _():
                pltpu.sync_copy(sh_part.at[:, pl.ds(sl * 128, 128)], red_in)

                def redloop(v, _):
                    dv = pl.ds(v * 16, 16)
                    t = red_in[0, dv]
                    for r in range(1, NSUB):
                        t = t + red_in[r, dv]
                    rsum[pl.ds(sl * 128 + v * 16, 16)] = t
                    return 0

                lax.fori_loop(0, 128 // 16, redloop, 0)
                pltpu.sync_copy(rsum.at[pl.ds(sl * 128, 128)],
                                sh_tot.at[pl.ds(sl * 128, 128)])

        plsc.subcore_barrier()
        pltpu.sync_copy(sh_tot, rsum)

        def vloop2(v, _):
            dv = pl.ds(v * 16, 16)
            r16 = rec_buf[dv]
            rinv = 1.0 / rsum[dv]
            for f in range(FPT):
                fv = jnp.full((16,), f, jnp.int32)
                en = eta_buf[f, dv] * rinv
                plsc.addupdate_scatter(s_tbl, [fv, r16], en)
            return 0

        lax.fori_loop(0, NV, vloop2, 0)
        if emit_eta:
            pltpu.sync_copy(eta_buf, eta_out.at[s, :, pl.ds(off, BLK)])

        return (a_s, a_q)

    zero16 = jnp.zeros((16,), jnp.float32)
    init = (tuple(zero16 for _ in range(FPT)), tuple(zero16 for _ in range(FPT)))
    a_s, a_q = lax.fori_loop(0, NBLKS, block_body, init)

    def zacc(i, _):
        zf = jnp.zeros((16,), jnp.float32)
        for f in range(FPT):
            acc_buf[f, pl.ds(i * 16, 16)] = zf
        return 0

    lax.fori_loop(0, 8, zacc, 0)
    for f in range(FPT):
        acc_buf[f, pl.ds(0, 16)] = a_s[f]
    pltpu.sync_copy(acc_buf, asum_out.at[co, s])
    for f in range(FPT):
        acc_buf[f, pl.ds(0, 16)] = a_q[f]
    pltpu.sync_copy(acc_buf, asq_out.at[co, s])
    pltpu.sync_copy(s_tbl, s_out.at[co, s])


def _make_k1(emit_eta):
    mesh = plsc.VectorSubcoreMesh(core_axis_name="c", subcore_axis_name="s",
                                  num_cores=NCORE, num_subcores=NSUB)
    outs = [
        jax.ShapeDtypeStruct((NCORE, NSUB, FPT, N), jnp.float32),
        jax.ShapeDtypeStruct((NCORE, NSUB, FPT, 128), jnp.float32),
        jax.ShapeDtypeStruct((NCORE, NSUB, FPT, 128), jnp.float32),
        jax.ShapeDtypeStruct((NSUB, FPT, E), jnp.float32),
    ]
    if not emit_eta:
        outs = outs[:-1]
    scratch = [
        pltpu.VMEM((FPT, N), jnp.float32),
        pltpu.VMEM((FPT, N), jnp.float32),
        pltpu.VMEM((BLK,), jnp.int32),
        pltpu.VMEM((BLK,), jnp.int32),
        pltpu.VMEM((FPT, BLK), jnp.float32),
        pltpu.VMEM((FPT, BLK), jnp.float32),
        pltpu.VMEM((BLK,), jnp.float32),
        pltpu.VMEM((NSUB, 128), jnp.float32),
        pltpu.VMEM((FPT, 128), jnp.float32),
        pltpu.VMEM_SHARED((NSUB, BLK), jnp.float32),
        pltpu.VMEM_SHARED((BLK,), jnp.float32),
    ]

    def body(*refs):
        if emit_eta:
            _k1_body(True, *refs)
        else:
            ins, rest = refs[:4], refs[4:]
            _k1_body(False, *ins, *rest[:3], None, *rest[3:])

    return pl.kernel(body, out_type=outs, mesh=mesh, scratch_types=scratch,
                     compiler_params=pltpu.CompilerParams(needs_layout_passes=False))


def _k0_body(send_hbm, cnt_out, jmax_out, send_buf, cnt_tbl, jmax_tbl):
    co = lax.axis_index("c")
    s = lax.axis_index("s")
    chunk = EHALF // NSUB
    base = co * EHALF + s * chunk
    pltpu.sync_copy(send_hbm.at[pl.ds(base, chunk)], send_buf.at[pl.ds(0, chunk)])

    def zloop(i, _):
        dv = pl.ds(i * 16, 16)
        cnt_tbl[dv] = jnp.zeros((16,), jnp.float32)
        jmax_tbl[dv] = jnp.full((16,), -1, jnp.int32)
        return 0

    lax.fori_loop(0, N // 16, zloop, 0)

    def jloop(v, _):
        col = v * 16
        dv = pl.ds(col, 16)
        lane = col + lax.iota(jnp.int32, 16)
        valid = lane < chunk
        j16 = jnp.where(valid, base + lane, -1)
        raw = send_buf[dv]
        idx = jnp.where(valid, raw, 0)
        plsc.addupdate_scatter(cnt_tbl, [idx], jnp.ones((16,), jnp.float32),
                               mask=valid)
        plsc.store_scatter(jmax_tbl, [idx], j16, mask=valid)
        g = plsc.load_gather(jmax_tbl, [idx])

        def wbody(active):
            g2 = plsc.load_gather(jmax_tbl, [idx])
            m = valid & (j16 > g2)
            plsc.store_scatter(jmax_tbl, [idx], j16, mask=m)
            g3 = plsc.load_gather(jmax_tbl, [idx])
            return jnp.any(valid & (j16 > g3))

        lax.while_loop(lambda a: a, wbody, jnp.any(valid & (j16 > g)))
        return 0

    lax.fori_loop(0, (chunk + 15) // 16, jloop, 0)
    pltpu.sync_copy(cnt_tbl, cnt_out.at[co, s])
    pltpu.sync_copy(jmax_tbl, jmax_out.at[co, s])


_k0_cache = {}


def _get_k0():
    if "k" not in _k0_cache:
        mesh = plsc.VectorSubcoreMesh(core_axis_name="c", subcore_axis_name="s",
                                      num_cores=NCORE, num_subcores=NSUB)
        chunk = EHALF // NSUB
        _k0_cache["k"] = pl.kernel(
            _k0_body,
            out_type=[
                jax.ShapeDtypeStruct((NCORE, NSUB, N), jnp.float32),
                jax.ShapeDtypeStruct((NCORE, NSUB, N), jnp.int32),
            ],
            mesh=mesh,
            scratch_types=[
                pltpu.VMEM((((chunk + 15) // 16) * 16,), jnp.int32),
                pltpu.VMEM((N,), jnp.float32),
                pltpu.VMEM((N,), jnp.int32),
            ],
            compiler_params=pltpu.CompilerParams(needs_layout_passes=False))
    return _k0_cache["k"]


_k1_cache = {}


def _get_k1(emit_eta):
    if emit_eta not in _k1_cache:
        _k1_cache[emit_eta] = _make_k1(emit_eta)
    return _k1_cache[emit_eta]


def _k2_body(vs_hbm, w_hbm, p2s_hbm, send_hbm, jc_hbm, mu_hbm,
             m_out, mp_out, x_out,
             tblA, tblB, idx1, xstage):
    co = lax.axis_index("c")
    s = lax.axis_index("s")

    @pl.when(co == 0)
    def _():
        pltpu.sync_copy(vs_hbm.at[s], tblA)
        pltpu.sync_copy(w_hbm.at[s], tblB)
        pltpu.sync_copy(send_hbm.at[pl.ds(0, N)], idx1)
        pltpu.sync_copy(mu_hbm.at[s], xstage)
        mu16 = [xstage[f, pl.ds(0, 16)] for f in range(FPT)]

        def xloop(v, carry):
            dv = pl.ds(v * 16, 16)
            s16 = idx1[dv]
            nc = []
            for f in range(FPT):
                fv = jnp.full((16,), f, jnp.int32)
                g = plsc.load_gather(tblA, [fv, s16])
                nc.append(carry[f] + (g - mu16[f]) * tblB[f, dv])
            return tuple(nc)

        z16 = jnp.zeros((16,), jnp.float32)
        xs = lax.fori_loop(0, N // 16, xloop, tuple(z16 for _ in range(FPT)))

        def zx(i, _):
            for f in range(FPT):
                xstage[f, pl.ds(i * 16, 16)] = jnp.zeros((16,), jnp.float32)
            return 0

        lax.fori_loop(0, 8, zx, 0)
        for f in range(FPT):
            xstage[f, pl.ds(0, 16)] = xs[f]
        pltpu.sync_copy(xstage, x_out.at[s])

        pltpu.sync_copy(p2s_hbm.at[s], tblA)
        pltpu.sync_copy(jc_hbm.at[pl.ds(0, N)], idx1)

        def mploop(v, _):
            dv = pl.ds(v * 16, 16)
            j16 = idx1[dv]
            for f in range(FPT):
                fv = jnp.full((16,), f, jnp.int32)
                tblB[f, dv] = plsc.load_gather(tblA, [fv, j16])
            return 0

        lax.fori_loop(0, N // 16, mploop, 0)
        pltpu.sync_copy(tblB, mp_out.at[s])

    @pl.when(co == 1)
    def _():
        pltpu.sync_copy(w_hbm.at[s], tblA)
        pltpu.sync_copy(jc_hbm.at[pl.ds(0, N)], idx1)

        def mloop(v, _):
            dv = pl.ds(v * 16, 16)
            j16 = idx1[dv]
            for f in range(FPT):
                fv = jnp.full((16,), f, jnp.int32)
                tblB[f, dv] = plsc.load_gather(tblA, [fv, j16])
            return 0

        lax.fori_loop(0, N // 16, mloop, 0)
        pltpu.sync_copy(tblB, m_out.at[s])


_k2_cache = {}


def _get_k2():
    if "k" not in _k2_cache:
        mesh = plsc.VectorSubcoreMesh(core_axis_name="c", subcore_axis_name="s",
                                      num_cores=NCORE, num_subcores=NSUB)
        _k2_cache["k"] = pl.kernel(
            _k2_body,
            out_type=[
                jax.ShapeDtypeStruct((NSUB, FPT, N), jnp.float32),
                jax.ShapeDtypeStruct((NSUB, FPT, N), jnp.float32),
                jax.ShapeDtypeStruct((NSUB, FPT, 128), jnp.float32),
            ],
            mesh=mesh,
            scratch_types=[
                pltpu.VMEM((FPT, N), jnp.float32),
                pltpu.VMEM((FPT, N), jnp.float32),
                pltpu.VMEM((N,), jnp.int32),
                pltpu.VMEM((FPT, 128), jnp.float32),
            ],
            compiler_params=pltpu.CompilerParams(needs_layout_passes=False))
    return _k2_cache["k"]


def _edge_pass(U3, ete3, send, rec, emit_eta):
    if emit_eta:
        S_part, asum, asq, eta = _get_k1(True)(U3, ete3, send, rec)
    else:
        S_part, asum, asq = _get_k1(False)(U3, ete3, send, rec)
        eta = None
    sum_eta = jnp.sum(asum, axis=(0, 3)).reshape(H)
    sumsq_eta = jnp.sum(asq, axis=(0, 3)).reshape(H)
    return S_part, sum_eta, sumsq_eta, eta


def _node_gather(Vs3, W3, P2S3, send, jc, Mu3):
    M3, Mp3, X_part = _get_k2()(Vs3, W3, P2S3, send, jc, Mu3)
    return jnp.sum(X_part, axis=2).reshape(H), M3, Mp3


def kernel(h, e, p, edge_index, batch, Wh, bh, We, be, Wp, bp, lin_W, lin_b,
           hps_W, hps_b, hpr_W, hpr_b, p1_W, p1_b, p2_W, p2_b, bn_w, bn_b,
           r1_W, r1_b, r2_W, r2_b, r3_W, r3_b):
    send = edge_index[0]
    rec = edge_index[1]
    ht, pt = _t0(h, Wh, bh, p, Wp, bp)
    cnt_part, jmax_part = _get_k0()(send)
    cnt, jc, upd, mm = _tj(cnt_part, jmax_part)

    eta0 = None
    bnm_e0 = None
    for l in range(L):
        U3, Vs3, Vr3, Q1_T, Q2_3 = _t1(
            ht, pt, lin_W[l], lin_b[l], hps_W[l], hps_b[l], hpr_W[l], hpr_b[l],
            p1_W[l], p1_b[l], p2_W[l], p2_b[l])
        if l == 0:
            ete3 = _te0(e, We, be, lin_W[0], lin_b[0])
        else:
            ete3 = _te1(e, eta0, We, be, lin_W[1], lin_b[1], bnm_e0)

        S_part, sum_eta, sumsq_eta, eta = _edge_pass(U3, ete3, send, rec, l == 0)
        if l == 0:
            eta0 = eta

        mu_e = 0.5 + sum_eta / E
        ce = sum_eta / E
        var_e = sumsq_eta / E - ce * ce
        inv_e = bn_w[l] / jnp.sqrt(var_e + 1e-5)
        if l == 0:
            bnm_e0 = jnp.stack([mu_e, inv_e, bn_b[l]], axis=0)

        W3, P2S3, RA, RB, RC, RD, Mu3 = _t2(S_part, Vr3, Q2_3, Vs3, cnt)
        X, M3, Mp3 = _node_gather(Vs3, W3, P2S3, send, jc.reshape(N), Mu3)

        mu_h = (RA.reshape(H) + RC.reshape(H)) / E
        var_h = (RB.reshape(H) + 2.0 * X + RD.reshape(H)) / E
        inv_h = bn_w[l] / jnp.sqrt(var_h + 1e-5)
        bnm_h = jnp.stack([mu_h, inv_h, bn_b[l]], axis=0)

        ht, pt = _t3(ht, pt, Vs3, M3, Q1_T, Mp3, bnm_h, upd, mm)

    hagg, pagg = _t4a(ht, pt, batch)
    out = _t4b(hagg, pagg, r1_W, r1_b, r2_W, r2_b, r3_W, r3_b)
    return out.reshape(G)
